# baseline (speedup 1.0000x reference)
import functools

import jax
import jax.numpy as jnp
from jax import lax
from jax.experimental import pallas as pl
from jax.experimental.pallas import tpu as pltpu
from jax.experimental.pallas import tpu_sc as plsc

NC = 2
NS = 16
NW = NC * NS
LANES = 16
CHUNK = 32
NBUF = 3


def _emb_ln_body(seq_len, hid, pb, pbc, nch,
                 ids_hbm, tok_hbm, pos_hbm, gamma_hbm, beta_hbm,
                 out_hbm, idx_v, rows_v, pos_v, g_v, b_v,
                 gsem, osem):
    nsl = hid // LANES
    wid = lax.axis_index("s") * NC + lax.axis_index("c")

    pltpu.sync_copy(ids_hbm.at[wid], idx_v)
    pltpu.sync_copy(gamma_hbm, g_v)
    pltpu.sync_copy(beta_hbm, b_v)
    pltpu.sync_copy(pos_hbm.at[pl.ds(wid * pb, pb)], pos_v)

    inv_hid = jnp.float32(1.0 / hid)

    def start_gather(k, s):
        pltpu.async_copy(tok_hbm.at[idx_v.at[k]], rows_v.at[s], gsem[s])

    def out_base(k):
        bb, h = k // pbc, k % pbc
        return bb * seq_len + wid * pb + h * CHUNK

    def wait_out(k):
        s = k % NBUF
        pltpu.make_async_copy(
            rows_v.at[s], out_hbm.at[pl.ds(out_base(k), CHUNK)], osem[s]).wait()

    start_gather(0, 0)
    start_gather(1, 1)

    for k in range(nch):
        s = k % NBUF
        h = k % pbc
        pltpu.make_async_copy(tok_hbm.at[idx_v.at[k]], rows_v.at[s],
                              gsem[s]).wait()

        def token_body(t, _):
            acc = jnp.zeros((LANES,), jnp.float32)
            acc2 = jnp.zeros((LANES,), jnp.float32)
            for j in range(nsl):
                sl = pl.ds(j * LANES, LANES)
                x = rows_v[s, t, sl] + pos_v[h * CHUNK + t, sl]
                rows_v[s, t, sl] = x
                acc = acc + x
                acc2 = acc2 + x * x
            mean = jnp.sum(acc) * inv_hid
            var = jnp.sum(acc2) * inv_hid - mean * mean
            vv = jnp.full((LANES,), var + jnp.float32(1e-12), jnp.float32)
            bits = plsc.bitcast(vv, jnp.int32)
            y = plsc.bitcast(jnp.int32(0x5F3759DF) - (bits >> 1), jnp.float32)
            for _ in range(3):
                y = y * (jnp.float32(1.5) - jnp.float32(0.5) * vv * y * y)
            meanv = jnp.full((LANES,), mean, jnp.float32)
            for j in range(nsl):
                sl = pl.ds(j * LANES, LANES)
                x = rows_v[s, t, sl]
                rows_v[s, t, sl] = (x - meanv) * y * g_v[sl] + b_v[sl]
            return ()

        lax.fori_loop(0, CHUNK, token_body, (), unroll=False)

        pltpu.async_copy(rows_v.at[s],
                         out_hbm.at[pl.ds(out_base(k), CHUNK)], osem[s])
        if k + 2 < nch:
            s2 = (k + 2) % NBUF
            if k >= 1:
                wait_out(k - 1)
            start_gather(k + 2, s2)

    for k in range(max(0, nch - NBUF), nch):
        wait_out(k)


def kernel(input_ids, tok_table, pos_table, gamma, beta):
    b, s = input_ids.shape
    vocab, hid = tok_table.shape
    tot = b * s
    pb = tot // NW // b
    pbc = pb // CHUNK
    nch = b * pbc

    ids = (input_ids.astype(jnp.int32)
           .reshape(b, NW, pbc, CHUNK)
           .transpose(1, 0, 2, 3)
           .reshape(NW, nch, CHUNK))

    mesh = plsc.VectorSubcoreMesh(core_axis_name="c", subcore_axis_name="s",
                                  num_cores=NC, num_subcores=NS)
    run = pl.kernel(
        functools.partial(_emb_ln_body, s, hid, pb, pbc, nch),
        out_type=jax.ShapeDtypeStruct((tot, hid), jnp.float32),
        mesh=mesh,
        scratch_types=[
            pltpu.VMEM((nch, CHUNK), jnp.int32),
            pltpu.VMEM((NBUF, CHUNK, hid), jnp.float32),
            pltpu.VMEM((pb, hid), jnp.float32),
            pltpu.VMEM((hid,), jnp.float32),
            pltpu.VMEM((hid,), jnp.float32),
            [pltpu.SemaphoreType.DMA] * NBUF,
            [pltpu.SemaphoreType.DMA] * NBUF,
        ],
        compiler_params=pltpu.CompilerParams(needs_layout_passes=False),
    )
    out = run(ids, tok_table, pos_table, gamma, beta)
    return out.reshape(b, s, hid)


# speedup vs baseline: 1.9920x; 1.9920x over previous
import functools

import jax
import jax.numpy as jnp
from jax import lax
from jax.experimental import pallas as pl
from jax.experimental.pallas import tpu as pltpu
from jax.experimental.pallas import tpu_sc as plsc

NC = 2
NS = 16
NW = NC * NS
CHUNK = 32
NBUF = 3


def _gather_body(tok_w, nch, ids_hbm, tok_hbm, gath_hbm, idx_v,
                 r0, r1, r2, gsem, osem):
    rows = [r0, r1, r2]
    wid = lax.axis_index("s") * NC + lax.axis_index("c")
    base = wid * tok_w

    pltpu.sync_copy(ids_hbm.at[wid], idx_v)

    def start_gather(k, s):
        pltpu.async_copy(tok_hbm.at[idx_v.at[k]], rows[s], gsem[s])

    def out_slice(k):
        return gath_hbm.at[pl.ds(base + k * CHUNK, CHUNK)]

    start_gather(0, 0)
    start_gather(1, 1)
    for k in range(nch):
        s = k % NBUF
        pltpu.make_async_copy(tok_hbm.at[idx_v.at[k]], rows[s], gsem[s]).wait()
        pltpu.async_copy(rows[s], out_slice(k), osem[s])
        if k + 2 < nch:
            s2 = (k + 2) % NBUF
            if k >= 1:
                pltpu.make_async_copy(rows[s2], out_slice(k - 1), osem[s2]).wait()
            start_gather(k + 2, s2)
    for k in range(max(0, nch - NBUF), nch):
        s = k % NBUF
        pltpu.make_async_copy(rows[s], out_slice(k), osem[s]).wait()


def _sc_gather(ids, tok_table):
    nw_tok = ids.shape[0] * ids.shape[1] * ids.shape[2] // NW
    nch = nw_tok // CHUNK
    hid = tok_table.shape[1]
    mesh = plsc.VectorSubcoreMesh(core_axis_name="c", subcore_axis_name="s",
                                  num_cores=NC, num_subcores=NS)
    run = pl.kernel(
        functools.partial(_gather_body, nw_tok, nch),
        out_type=jax.ShapeDtypeStruct((NW * nw_tok, hid), jnp.float32),
        mesh=mesh,
        scratch_types=[
            pltpu.VMEM((nch, CHUNK), jnp.int32),
            pltpu.VMEM((CHUNK, hid), jnp.float32),
            pltpu.VMEM((CHUNK, hid), jnp.float32),
            pltpu.VMEM((CHUNK, hid), jnp.float32),
            [pltpu.SemaphoreType.DMA] * NBUF,
            [pltpu.SemaphoreType.DMA] * NBUF,
        ],
        compiler_params=pltpu.CompilerParams(needs_layout_passes=False),
    )
    return run(ids, tok_table)


def _ln_block(emb_ref, pos_ref, g_ref, b_ref, out_ref):
    x = emb_ref[...] + pos_ref[...]
    mean = jnp.mean(x, axis=1, keepdims=True)
    xc = x - mean
    var = jnp.mean(xc * xc, axis=1, keepdims=True)
    y = xc * lax.rsqrt(var + 1e-12)
    out_ref[...] = y * g_ref[...] + b_ref[...]


def _tc_layernorm(emb, pos_table, gamma, beta, blk):
    tot, hid = emb.shape
    s = pos_table.shape[0]
    bps = s // blk
    grid = (tot // blk,)
    return pl.pallas_call(
        _ln_block,
        grid=grid,
        in_specs=[
            pl.BlockSpec((blk, hid), lambda i: (i, 0)),
            pl.BlockSpec((blk, hid), lambda i: (lax.rem(i, bps), 0)),
            pl.BlockSpec((1, hid), lambda i: (0, 0)),
            pl.BlockSpec((1, hid), lambda i: (0, 0)),
        ],
        out_specs=pl.BlockSpec((blk, hid), lambda i: (i, 0)),
        out_shape=jax.ShapeDtypeStruct((tot, hid), jnp.float32),
    )(emb, pos_table, gamma.reshape(1, hid), beta.reshape(1, hid))


def kernel(input_ids, tok_table, pos_table, gamma, beta):
    b, s = input_ids.shape
    hid = tok_table.shape[1]
    tot = b * s
    tok_w = tot // NW
    nch = tok_w // CHUNK

    ids = input_ids.astype(jnp.int32).reshape(NW, nch, CHUNK)
    emb = _sc_gather(ids, tok_table)
    out = _tc_layernorm(emb, pos_table, gamma, beta, 256)
    return out.reshape(b, s, hid)
